# baseline (speedup 1.0000x reference)
import functools

import jax
import jax.numpy as jnp
from jax import lax
from jax.experimental import pallas as pl
from jax.experimental.pallas import tpu as pltpu
from jax.experimental.pallas import tpu_sc as plsc

N = 10000
D = 128
E = 320000

NC = 2
NS = 16
NW = NC * NS
K = 128
CPB = 4
NBLK = 20
NBLK_T = NBLK + 2
E_PAD = NW * NBLK * CPB * K
ROWS_PT = 632
ACC_ROWS = NS * ROWS_PT


def _sc_agg_body(x_hbm, idx_hbm, zero_hbm, out_hbm, slot0, slot1, rows0, rows1,
                 acc, isem, gsem0, gsem1):
    c = lax.axis_index("c")
    s = lax.axis_index("s")
    w = c * NS + s

    pltpu.sync_copy(zero_hbm, acc.at[pl.ds(s * ROWS_PT, ROWS_PT)])
    plsc.subcore_barrier()

    slots = (slot0, slot1)
    rows = (rows0, rows1)

    def i_start(b, q):
        pltpu.async_copy(idx_hbm.at[w, b], slots[q], isem)

    def i_wait(q):
        pltpu.make_async_copy(idx_hbm.at[w, 0], slots[q], isem).wait()

    gsems = (gsem0, gsem1)

    def g_start(q, m, r):
        pltpu.async_copy(x_hbm.at[slots[q].at[m]], rows[r], gsems[r])

    def g_wait(r):
        pltpu.make_async_copy(x_hbm.at[slots[0].at[0]], rows[r],
                              gsems[r]).wait()

    def scat(q, m, r):
        pltpu.sync_copy(rows[r], acc.at[slots[q].at[CPB + m]], add=True)

    i_start(0, 0)
    i_start(1, 1)
    i_wait(0)
    g_start(0, 0, 0)
    g_start(0, 1, 1)

    def body(i, carry):
        b0 = 2 * i
        for q in (0, 1):
            for m in range(2):
                g_wait(m % 2)
                scat(q, m, m % 2)
                g_start(q, m + 2, m % 2)
            g_wait(0)
            scat(q, 2, 0)
            i_wait(1 - q)
            g_start(1 - q, 0, 0)
            g_wait(1)
            scat(q, 3, 1)
            g_start(1 - q, 1, 1)
            i_start(b0 + q + 2, q)
        return carry

    lax.fori_loop(0, NBLK // 2, body, 0)
    g_wait(0)
    g_wait(1)
    i_wait(1)
    plsc.subcore_barrier()

    pltpu.sync_copy(acc.at[pl.ds(s * ROWS_PT, ROWS_PT)],
                    out_hbm.at[c, pl.ds(s * ROWS_PT, ROWS_PT)])


@functools.cache
def _get_sc_agg():
    return functools.partial(
        pl.kernel,
        out_type=jax.ShapeDtypeStruct((NC, ACC_ROWS, D), jnp.float32),
        mesh=plsc.VectorSubcoreMesh(core_axis_name="c", subcore_axis_name="s",
                                    num_cores=NC, num_subcores=NS),
        scratch_types=[
            pltpu.VMEM((2 * CPB, K), jnp.int32),
            pltpu.VMEM((2 * CPB, K), jnp.int32),
            pltpu.VMEM((K, D), jnp.float32),
            pltpu.VMEM((K, D), jnp.float32),
            pltpu.VMEM_SHARED((ACC_ROWS, D), jnp.float32),
            pltpu.SemaphoreType.DMA,
            pltpu.SemaphoreType.DMA,
            pltpu.SemaphoreType.DMA,
        ],
    )(_sc_agg_body)


def _mlp_body(eps_ref, x_ref, p0_ref, p1_ref, wa_ref, ba_ref, g_ref, be_ref,
              wb_ref, bb_ref, o_ref, *, final_act):
    h0 = x_ref[...] * (1.0 + eps_ref[0]) + p0_ref[0] + p1_ref[0]
    h = lax.dot_general(h0, wa_ref[...], (((1,), (0,)), ((), ())),
                        precision=lax.Precision.HIGHEST,
                        preferred_element_type=jnp.float32) + ba_ref[...]
    mu = jnp.mean(h, axis=-1, keepdims=True)
    var = jnp.mean((h - mu) ** 2, axis=-1, keepdims=True)
    h = (h - mu) / jnp.sqrt(var + 1e-5) * g_ref[...] + be_ref[...]
    h = jnp.where(h > 0, h, 0.01 * h)
    h = lax.dot_general(h, wb_ref[...], (((1,), (0,)), ((), ())),
                        precision=lax.Precision.HIGHEST,
                        preferred_element_type=jnp.float32) + bb_ref[...]
    nrm = jnp.sqrt(jnp.sum(h * h, axis=-1, keepdims=True))
    h = h / jnp.maximum(nrm, 1e-12)
    if final_act:
        h = jnp.where(h > 0, h, 0.01 * h)
    o_ref[...] = h


def _tc_layer(eps, x, parts, waT, ba, g, be, wbT, bb, final_act, interpret=False):
    R = 2000
    grid = (N // R,)
    row_spec = pl.BlockSpec((R, D), lambda i: (i, 0))
    part0_spec = pl.BlockSpec((1, R, D), lambda i: (0, i, 0))
    part1_spec = pl.BlockSpec((1, R, D), lambda i: (1, i, 0))
    full_spec = pl.BlockSpec((D, D), lambda i: (0, 0))
    vec_spec = pl.BlockSpec((1, D), lambda i: (0, 0))
    return pl.pallas_call(
        functools.partial(_mlp_body, final_act=final_act),
        grid=grid,
        in_specs=[
            pl.BlockSpec(memory_space=pltpu.SMEM),
            row_spec, part0_spec, part1_spec,
            full_spec, vec_spec, vec_spec, vec_spec, full_spec, vec_spec,
        ],
        out_specs=row_spec,
        out_shape=jax.ShapeDtypeStruct((N, D), jnp.float32),
        interpret=interpret,
    )(eps, x, parts, parts, waT, ba, g, be, wbT, bb)


def kernel(x, edge_index, eps1, W1a, b1a, g1, be1, W1b, b1b,
           eps2, W2a, b2a, g2, be2, W2b, b2b):
    src = edge_index[0]
    dst = edge_index[1]
    npad = E_PAD - E
    srcp = jnp.concatenate(
        [src, jnp.zeros((npad,), jnp.int32)]).reshape(NW, NBLK, CPB, K)
    dstp = jnp.concatenate(
        [dst, jnp.full((npad,), N, jnp.int32)]).reshape(NW, NBLK, CPB, K)
    src_all = jnp.concatenate(
        [srcp, jnp.zeros((NW, 2, CPB, K), jnp.int32)], axis=1)
    dst_all = jnp.concatenate(
        [dstp, jnp.full((NW, 2, CPB, K), N, jnp.int32)], axis=1)
    idx_all = jnp.concatenate([src_all, dst_all], axis=2)
    zero_rows = jnp.zeros((ROWS_PT, D), jnp.float32)

    e1 = jnp.reshape(eps1, (1,))
    e2 = jnp.reshape(eps2, (1,))
    vecs = [v.reshape(1, D) for v in (b1a, g1, be1, b1b, b2a, g2, be2, b2b)]
    b1a_, g1_, be1_, b1b_, b2a_, g2_, be2_, b2b_ = vecs

    sc_agg = _get_sc_agg()
    parts1 = sc_agg(x, idx_all, zero_rows)
    h1 = _tc_layer(e1, x, parts1, W1a.T, b1a_, g1_, be1_, W1b.T, b1b_,
                   final_act=True)
    parts2 = sc_agg(h1, idx_all, zero_rows)
    h2 = _tc_layer(e2, h1, parts2, W2a.T, b2a_, g2_, be2_, W2b.T, b2b_,
                   final_act=False)
    return h2


# speedup vs baseline: 1.5503x; 1.5503x over previous
import functools

import jax
import jax.numpy as jnp
from jax import lax
from jax.experimental import pallas as pl
from jax.experimental.pallas import tpu as pltpu
from jax.experimental.pallas import tpu_sc as plsc

N = 10000
D = 128
E = 320000

NC = 2
NS = 16
NW = NC * NS
K = 128
CH = 80
PAIRS = CH // 2
T_PAD = PAIRS + 2
E_PAD = NW * CH * K
ROWS_PT = 632
ACC_ROWS = NS * ROWS_PT


def _sc_agg_body(x_hbm, src_hbm, dst_hbm, zero_hbm, out_hbm, src_v,
                 dslot0, dslot1, rows0, rows1, acc,
                 dsem0, dsem1, gsem0, gsem1):
    c = lax.axis_index("c")
    s = lax.axis_index("s")
    w = c * NS + s

    pltpu.sync_copy(zero_hbm, acc.at[pl.ds(s * ROWS_PT, ROWS_PT)])
    plsc.subcore_barrier()

    pltpu.sync_copy(src_hbm.at[w], src_v)

    dslots = (dslot0, dslot1)
    dsems = (dsem0, dsem1)
    rows = (rows0, rows1)
    gsems = (gsem0, gsem1)

    def d_start(q, t):
        pltpu.async_copy(dst_hbm.at[w, t], dslots[q], dsems[q])

    def d_wait(q):
        pltpu.make_async_copy(dst_hbm.at[w, 0], dslots[q], dsems[q]).wait()

    def g_start(j, r):
        return pltpu.async_copy(x_hbm.at[src_v.at[j]], rows[r], gsems[r])

    def scat(q, m, r):
        pltpu.sync_copy(rows[r], acc.at[dslots[q].at[m]], add=True)

    d_start(0, 0)
    d_start(1, 1)

    def body(i, carry):
        j = 4 * i
        d_wait(0)
        dg0 = g_start(j, 0)
        dg1 = g_start(j + 1, 1)
        d_wait(1)
        dg0.wait()
        scat(0, 0, 0)
        dg2 = g_start(j + 2, 0)
        dg1.wait()
        scat(0, 1, 1)
        dg3 = g_start(j + 3, 1)
        d_start(0, 2 * i + 2)
        dg2.wait()
        scat(1, 0, 0)
        dg3.wait()
        scat(1, 1, 1)
        d_start(1, 2 * i + 3)
        return carry

    lax.fori_loop(0, CH // 4, body, 0)
    d_wait(0)
    d_wait(1)
    plsc.subcore_barrier()

    pltpu.sync_copy(acc.at[pl.ds(s * ROWS_PT, ROWS_PT)],
                    out_hbm.at[c, pl.ds(s * ROWS_PT, ROWS_PT)])


@functools.cache
def _get_sc_agg():
    return functools.partial(
        pl.kernel,
        out_type=jax.ShapeDtypeStruct((NC, ACC_ROWS, D), jnp.float32),
        mesh=plsc.VectorSubcoreMesh(core_axis_name="c", subcore_axis_name="s",
                                    num_cores=NC, num_subcores=NS),
        scratch_types=[
            pltpu.VMEM((CH, K), jnp.int32),
            pltpu.VMEM((2, K), jnp.int32),
            pltpu.VMEM((2, K), jnp.int32),
            pltpu.VMEM((K, D), jnp.float32),
            pltpu.VMEM((K, D), jnp.float32),
            pltpu.VMEM_SHARED((ACC_ROWS, D), jnp.float32),
            pltpu.SemaphoreType.DMA,
            pltpu.SemaphoreType.DMA,
            pltpu.SemaphoreType.DMA,
            pltpu.SemaphoreType.DMA,
        ],
    )(_sc_agg_body)


def _mlp_body(eps_ref, x_ref, p0_ref, p1_ref, wa_ref, ba_ref, g_ref, be_ref,
              wb_ref, bb_ref, o_ref, *, final_act):
    h0 = x_ref[...] * (1.0 + eps_ref[0]) + p0_ref[0] + p1_ref[0]
    h = lax.dot_general(h0, wa_ref[...], (((1,), (0,)), ((), ())),
                        precision=lax.Precision.HIGHEST,
                        preferred_element_type=jnp.float32) + ba_ref[...]
    mu = jnp.mean(h, axis=-1, keepdims=True)
    var = jnp.mean((h - mu) ** 2, axis=-1, keepdims=True)
    h = (h - mu) / jnp.sqrt(var + 1e-5) * g_ref[...] + be_ref[...]
    h = jnp.where(h > 0, h, 0.01 * h)
    h = lax.dot_general(h, wb_ref[...], (((1,), (0,)), ((), ())),
                        precision=lax.Precision.HIGHEST,
                        preferred_element_type=jnp.float32) + bb_ref[...]
    nrm = jnp.sqrt(jnp.sum(h * h, axis=-1, keepdims=True))
    h = h / jnp.maximum(nrm, 1e-12)
    if final_act:
        h = jnp.where(h > 0, h, 0.01 * h)
    o_ref[...] = h


def _tc_layer(eps, x, parts, waT, ba, g, be, wbT, bb, final_act, interpret=False):
    R = 2000
    grid = (N // R,)
    row_spec = pl.BlockSpec((R, D), lambda i: (i, 0))
    part0_spec = pl.BlockSpec((1, R, D), lambda i: (0, i, 0))
    part1_spec = pl.BlockSpec((1, R, D), lambda i: (1, i, 0))
    full_spec = pl.BlockSpec((D, D), lambda i: (0, 0))
    vec_spec = pl.BlockSpec((1, D), lambda i: (0, 0))
    return pl.pallas_call(
        functools.partial(_mlp_body, final_act=final_act),
        grid=grid,
        in_specs=[
            pl.BlockSpec(memory_space=pltpu.SMEM),
            row_spec, part0_spec, part1_spec,
            full_spec, vec_spec, vec_spec, vec_spec, full_spec, vec_spec,
        ],
        out_specs=row_spec,
        out_shape=jax.ShapeDtypeStruct((N, D), jnp.float32),
        interpret=interpret,
    )(eps, x, parts, parts, waT, ba, g, be, wbT, bb)


def kernel(x, edge_index, eps1, W1a, b1a, g1, be1, W1b, b1b,
           eps2, W2a, b2a, g2, be2, W2b, b2b):
    src = edge_index[0]
    dst = edge_index[1]
    npad = E_PAD - E
    src_all = jnp.concatenate(
        [src, jnp.zeros((npad,), jnp.int32)]).reshape(NW, CH, K)
    dst_all = jnp.concatenate(
        [dst, jnp.full((npad,), N, jnp.int32)]).reshape(NW, PAIRS, 2, K)
    dst_all = jnp.concatenate(
        [dst_all, jnp.full((NW, 2, 2, K), N, jnp.int32)], axis=1)
    zero_rows = jnp.zeros((ROWS_PT, D), jnp.float32)

    e1 = jnp.reshape(eps1, (1,))
    e2 = jnp.reshape(eps2, (1,))
    vecs = [v.reshape(1, D) for v in (b1a, g1, be1, b1b, b2a, g2, be2, b2b)]
    b1a_, g1_, be1_, b1b_, b2a_, g2_, be2_, b2b_ = vecs

    sc_agg = _get_sc_agg()
    parts1 = sc_agg(x, src_all, dst_all, zero_rows)
    h1 = _tc_layer(e1, x, parts1, W1a.T, b1a_, g1_, be1_, W1b.T, b1b_,
                   final_act=True)
    parts2 = sc_agg(h1, src_all, dst_all, zero_rows)
    h2 = _tc_layer(e2, h1, parts2, W2a.T, b2a_, g2_, be2_, W2b.T, b2b_,
                   final_act=False)
    return h2


# speedup vs baseline: 1.8520x; 1.1946x over previous
import functools

import jax
import jax.numpy as jnp
from jax import lax
from jax.experimental import pallas as pl
from jax.experimental.pallas import tpu as pltpu
from jax.experimental.pallas import tpu_sc as plsc

N = 10000
D = 128
E = 320000

NC = 2
NS = 16
NW = NC * NS
K = 128
CPB = 4
FAST_C = 0
F_BLK = 32
S_BLK = 8
F_T = F_BLK + 2
S_T = S_BLK + 2
TOTBLK = NS * (F_T + S_T)
E_PAD = NS * (F_BLK + S_BLK) * CPB * K
ROWS_PT = 632
ACC_ROWS = NS * ROWS_PT


def _sc_agg_body(x_hbm, blk_hbm, zero_hbm, out_hbm,
                 slot0, slot1, rows0, rows1, acc,
                 ssem0, ssem1, gsem0, gsem1):
    c = lax.axis_index("c")
    s = lax.axis_index("s")

    pltpu.sync_copy(zero_hbm, acc.at[pl.ds(s * ROWS_PT, ROWS_PT)])
    plsc.subcore_barrier()

    base = jnp.where(c == FAST_C, s * F_T, NS * F_T + s * S_T)
    nbody = jnp.where(c == FAST_C, F_BLK // 2, S_BLK // 2)

    slots = (slot0, slot1)
    ssems = (ssem0, ssem1)
    rows = (rows0, rows1)
    gsems = (gsem0, gsem1)

    def sl_start(q, t):
        pltpu.async_copy(blk_hbm.at[t], slots[q], ssems[q])

    def sl_wait(q):
        pltpu.make_async_copy(blk_hbm.at[0], slots[q], ssems[q]).wait()

    def g_start(q, m, r):
        return pltpu.async_copy(x_hbm.at[slots[q].at[m]], rows[r], gsems[r])

    def scat(q, m, r):
        pltpu.sync_copy(rows[r], acc.at[slots[q].at[CPB + m]], add=True)

    sl_start(0, base)
    sl_start(1, base + 1)

    def body(i, carry):
        t = base + 2 * i
        sl_wait(0)
        d0 = g_start(0, 0, 0)
        d1 = g_start(0, 1, 1)
        d0.wait()
        scat(0, 0, 0)
        d2 = g_start(0, 2, 0)
        d1.wait()
        scat(0, 1, 1)
        d3 = g_start(0, 3, 1)
        sl_wait(1)
        d2.wait()
        scat(0, 2, 0)
        d4 = g_start(1, 0, 0)
        d3.wait()
        scat(0, 3, 1)
        d5 = g_start(1, 1, 1)
        sl_start(0, t + 2)
        d4.wait()
        scat(1, 0, 0)
        d6 = g_start(1, 2, 0)
        d5.wait()
        scat(1, 1, 1)
        d7 = g_start(1, 3, 1)
        d6.wait()
        scat(1, 2, 0)
        d7.wait()
        scat(1, 3, 1)
        sl_start(1, t + 3)
        return carry

    lax.fori_loop(0, nbody, body, 0)
    sl_wait(0)
    sl_wait(1)
    plsc.subcore_barrier()

    pltpu.sync_copy(acc.at[pl.ds(s * ROWS_PT, ROWS_PT)],
                    out_hbm.at[c, pl.ds(s * ROWS_PT, ROWS_PT)])


@functools.cache
def _get_sc_agg():
    return functools.partial(
        pl.kernel,
        out_type=jax.ShapeDtypeStruct((NC, ACC_ROWS, D), jnp.float32),
        mesh=plsc.VectorSubcoreMesh(core_axis_name="c", subcore_axis_name="s",
                                    num_cores=NC, num_subcores=NS),
        scratch_types=[
            pltpu.VMEM((2 * CPB, K), jnp.int32),
            pltpu.VMEM((2 * CPB, K), jnp.int32),
            pltpu.VMEM((K, D), jnp.float32),
            pltpu.VMEM((K, D), jnp.float32),
            pltpu.VMEM_SHARED((ACC_ROWS, D), jnp.float32),
            pltpu.SemaphoreType.DMA,
            pltpu.SemaphoreType.DMA,
            pltpu.SemaphoreType.DMA,
            pltpu.SemaphoreType.DMA,
        ],
    )(_sc_agg_body)


def _mlp_body(eps_ref, x_ref, p0_ref, p1_ref, wa_ref, ba_ref, g_ref, be_ref,
              wb_ref, bb_ref, o_ref, *, final_act):
    h0 = x_ref[...] * (1.0 + eps_ref[0]) + p0_ref[0] + p1_ref[0]
    h = lax.dot_general(h0, wa_ref[...], (((1,), (0,)), ((), ())),
                        precision=lax.Precision.HIGHEST,
                        preferred_element_type=jnp.float32) + ba_ref[...]
    mu = jnp.mean(h, axis=-1, keepdims=True)
    var = jnp.mean((h - mu) ** 2, axis=-1, keepdims=True)
    h = (h - mu) / jnp.sqrt(var + 1e-5) * g_ref[...] + be_ref[...]
    h = jnp.where(h > 0, h, 0.01 * h)
    h = lax.dot_general(h, wb_ref[...], (((1,), (0,)), ((), ())),
                        precision=lax.Precision.HIGHEST,
                        preferred_element_type=jnp.float32) + bb_ref[...]
    nrm = jnp.sqrt(jnp.sum(h * h, axis=-1, keepdims=True))
    h = h / jnp.maximum(nrm, 1e-12)
    if final_act:
        h = jnp.where(h > 0, h, 0.01 * h)
    o_ref[...] = h


def _tc_layer(eps, x, parts, waT, ba, g, be, wbT, bb, final_act, interpret=False):
    R = 2000
    grid = (N // R,)
    row_spec = pl.BlockSpec((R, D), lambda i: (i, 0))
    part0_spec = pl.BlockSpec((1, R, D), lambda i: (0, i, 0))
    part1_spec = pl.BlockSpec((1, R, D), lambda i: (1, i, 0))
    full_spec = pl.BlockSpec((D, D), lambda i: (0, 0))
    vec_spec = pl.BlockSpec((1, D), lambda i: (0, 0))
    return pl.pallas_call(
        functools.partial(_mlp_body, final_act=final_act),
        grid=grid,
        in_specs=[
            pl.BlockSpec(memory_space=pltpu.SMEM),
            row_spec, part0_spec, part1_spec,
            full_spec, vec_spec, vec_spec, vec_spec, full_spec, vec_spec,
        ],
        out_specs=row_spec,
        out_shape=jax.ShapeDtypeStruct((N, D), jnp.float32),
        interpret=interpret,
    )(eps, x, parts, parts, waT, ba, g, be, wbT, bb)


def kernel(x, edge_index, eps1, W1a, b1a, g1, be1, W1b, b1b,
           eps2, W2a, b2a, g2, be2, W2b, b2b):
    src = edge_index[0]
    dst = edge_index[1]
    npad = E_PAD - E
    srcp = jnp.concatenate([src, jnp.zeros((npad,), jnp.int32)])
    dstp = jnp.concatenate([dst, jnp.full((npad,), N, jnp.int32)])
    fast_e = NS * F_BLK * CPB * K

    def mk_blocks(flat_s, flat_d, nblk, nblk_t):
        s3 = flat_s.reshape(NS, nblk, CPB, K)
        d3 = flat_d.reshape(NS, nblk, CPB, K)
        blk = jnp.concatenate([s3, d3], axis=2)
        dummy = jnp.concatenate(
            [jnp.zeros((NS, nblk_t - nblk, CPB, K), jnp.int32),
             jnp.full((NS, nblk_t - nblk, CPB, K), N, jnp.int32)], axis=2)
        return jnp.concatenate([blk, dummy], axis=1).reshape(-1, 2 * CPB, K)

    blk_all = jnp.concatenate([
        mk_blocks(srcp[:fast_e], dstp[:fast_e], F_BLK, F_T),
        mk_blocks(srcp[fast_e:], dstp[fast_e:], S_BLK, S_T),
    ])
    zero_rows = jnp.zeros((ROWS_PT, D), jnp.float32)

    e1 = jnp.reshape(eps1, (1,))
    e2 = jnp.reshape(eps2, (1,))
    vecs = [v.reshape(1, D) for v in (b1a, g1, be1, b1b, b2a, g2, be2, b2b)]
    b1a_, g1_, be1_, b1b_, b2a_, g2_, be2_, b2b_ = vecs

    sc_agg = _get_sc_agg()
    parts1 = sc_agg(x, blk_all, zero_rows)
    h1 = _tc_layer(e1, x, parts1, W1a.T, b1a_, g1_, be1_, W1b.T, b1b_,
                   final_act=True)
    parts2 = sc_agg(h1, blk_all, zero_rows)
    h2 = _tc_layer(e2, h1, parts2, W2a.T, b2a_, g2_, be2_, W2b.T, b2b_,
                   final_act=False)
    return h2


# speedup vs baseline: 2.0001x; 1.0800x over previous
import functools

import jax
import jax.numpy as jnp
from jax import lax
from jax.experimental import pallas as pl
from jax.experimental.pallas import tpu as pltpu
from jax.experimental.pallas import tpu_sc as plsc

N = 10000
D = 128
E = 320000

NC = 2
NS = 16
NW = NC * NS
K = 128
CPB = 4
FAST_C = 0
F_BLK = 36
S_BLK = 4
F_T = F_BLK + 2
S_T = S_BLK + 2
TOTBLK = NS * (F_T + S_T)
E_PAD = NS * (F_BLK + S_BLK) * CPB * K
ROWS_PT = 632
ACC_ROWS = NS * ROWS_PT


def _sc_agg_body(x_hbm, blk_hbm, zero_hbm, out_hbm,
                 slot0, slot1, rows0, rows1, acc,
                 ssem0, ssem1, gsem0, gsem1):
    c = lax.axis_index("c")
    s = lax.axis_index("s")

    pltpu.sync_copy(zero_hbm, acc.at[pl.ds(s * ROWS_PT, ROWS_PT)])
    plsc.subcore_barrier()

    base = jnp.where(c == FAST_C, s * F_T, NS * F_T + s * S_T)
    nbody = jnp.where(c == FAST_C, F_BLK // 2, S_BLK // 2)

    slots = (slot0, slot1)
    ssems = (ssem0, ssem1)
    rows = (rows0, rows1)
    gsems = (gsem0, gsem1)

    def sl_start(q, t):
        pltpu.async_copy(blk_hbm.at[t], slots[q], ssems[q])

    def sl_wait(q):
        pltpu.make_async_copy(blk_hbm.at[0], slots[q], ssems[q]).wait()

    def g_start(q, m, r):
        return pltpu.async_copy(x_hbm.at[slots[q].at[m]], rows[r], gsems[r])

    def scat(q, m, r):
        pltpu.sync_copy(rows[r], acc.at[slots[q].at[CPB + m]], add=True)

    sl_start(0, base)
    sl_start(1, base + 1)

    def body(i, carry):
        t = base + 2 * i
        sl_wait(0)
        d0 = g_start(0, 0, 0)
        d1 = g_start(0, 1, 1)
        d0.wait()
        scat(0, 0, 0)
        d2 = g_start(0, 2, 0)
        d1.wait()
        scat(0, 1, 1)
        d3 = g_start(0, 3, 1)
        sl_wait(1)
        d2.wait()
        scat(0, 2, 0)
        d4 = g_start(1, 0, 0)
        d3.wait()
        scat(0, 3, 1)
        d5 = g_start(1, 1, 1)
        sl_start(0, t + 2)
        d4.wait()
        scat(1, 0, 0)
        d6 = g_start(1, 2, 0)
        d5.wait()
        scat(1, 1, 1)
        d7 = g_start(1, 3, 1)
        d6.wait()
        scat(1, 2, 0)
        d7.wait()
        scat(1, 3, 1)
        sl_start(1, t + 3)
        return carry

    lax.fori_loop(0, nbody, body, 0)
    sl_wait(0)
    sl_wait(1)
    plsc.subcore_barrier()

    pltpu.sync_copy(acc.at[pl.ds(s * ROWS_PT, ROWS_PT)],
                    out_hbm.at[c, pl.ds(s * ROWS_PT, ROWS_PT)])


@functools.cache
def _get_sc_agg():
    return functools.partial(
        pl.kernel,
        out_type=jax.ShapeDtypeStruct((NC, ACC_ROWS, D), jnp.float32),
        mesh=plsc.VectorSubcoreMesh(core_axis_name="c", subcore_axis_name="s",
                                    num_cores=NC, num_subcores=NS),
        scratch_types=[
            pltpu.VMEM((2 * CPB, K), jnp.int32),
            pltpu.VMEM((2 * CPB, K), jnp.int32),
            pltpu.VMEM((K, D), jnp.float32),
            pltpu.VMEM((K, D), jnp.float32),
            pltpu.VMEM_SHARED((ACC_ROWS, D), jnp.float32),
            pltpu.SemaphoreType.DMA,
            pltpu.SemaphoreType.DMA,
            pltpu.SemaphoreType.DMA,
            pltpu.SemaphoreType.DMA,
        ],
    )(_sc_agg_body)


def _mlp_body(eps_ref, x_ref, p0_ref, p1_ref, wa_ref, ba_ref, g_ref, be_ref,
              wb_ref, bb_ref, o_ref, *, final_act):
    h0 = x_ref[...] * (1.0 + eps_ref[0]) + p0_ref[0] + p1_ref[0]
    h = lax.dot_general(h0, wa_ref[...], (((1,), (0,)), ((), ())),
                        precision=lax.Precision.HIGHEST,
                        preferred_element_type=jnp.float32) + ba_ref[...]
    mu = jnp.mean(h, axis=-1, keepdims=True)
    var = jnp.mean((h - mu) ** 2, axis=-1, keepdims=True)
    h = (h - mu) / jnp.sqrt(var + 1e-5) * g_ref[...] + be_ref[...]
    h = jnp.where(h > 0, h, 0.01 * h)
    h = lax.dot_general(h, wb_ref[...], (((1,), (0,)), ((), ())),
                        precision=lax.Precision.HIGHEST,
                        preferred_element_type=jnp.float32) + bb_ref[...]
    nrm = jnp.sqrt(jnp.sum(h * h, axis=-1, keepdims=True))
    h = h / jnp.maximum(nrm, 1e-12)
    if final_act:
        h = jnp.where(h > 0, h, 0.01 * h)
    o_ref[...] = h


def _tc_layer(eps, x, parts, waT, ba, g, be, wbT, bb, final_act, interpret=False):
    R = 2000
    grid = (N // R,)
    row_spec = pl.BlockSpec((R, D), lambda i: (i, 0))
    part0_spec = pl.BlockSpec((1, R, D), lambda i: (0, i, 0))
    part1_spec = pl.BlockSpec((1, R, D), lambda i: (1, i, 0))
    full_spec = pl.BlockSpec((D, D), lambda i: (0, 0))
    vec_spec = pl.BlockSpec((1, D), lambda i: (0, 0))
    return pl.pallas_call(
        functools.partial(_mlp_body, final_act=final_act),
        grid=grid,
        in_specs=[
            pl.BlockSpec(memory_space=pltpu.SMEM),
            row_spec, part0_spec, part1_spec,
            full_spec, vec_spec, vec_spec, vec_spec, full_spec, vec_spec,
        ],
        out_specs=row_spec,
        out_shape=jax.ShapeDtypeStruct((N, D), jnp.float32),
        interpret=interpret,
    )(eps, x, parts, parts, waT, ba, g, be, wbT, bb)


def kernel(x, edge_index, eps1, W1a, b1a, g1, be1, W1b, b1b,
           eps2, W2a, b2a, g2, be2, W2b, b2b):
    src = edge_index[0]
    dst = edge_index[1]
    npad = E_PAD - E
    srcp = jnp.concatenate([src, jnp.zeros((npad,), jnp.int32)])
    dstp = jnp.concatenate([dst, jnp.full((npad,), N, jnp.int32)])
    fast_e = NS * F_BLK * CPB * K

    def mk_blocks(flat_s, flat_d, nblk, nblk_t):
        s3 = flat_s.reshape(NS, nblk, CPB, K)
        d3 = flat_d.reshape(NS, nblk, CPB, K)
        blk = jnp.concatenate([s3, d3], axis=2)
        dummy = jnp.concatenate(
            [jnp.zeros((NS, nblk_t - nblk, CPB, K), jnp.int32),
             jnp.full((NS, nblk_t - nblk, CPB, K), N, jnp.int32)], axis=2)
        return jnp.concatenate([blk, dummy], axis=1).reshape(-1, 2 * CPB, K)

    blk_all = jnp.concatenate([
        mk_blocks(srcp[:fast_e], dstp[:fast_e], F_BLK, F_T),
        mk_blocks(srcp[fast_e:], dstp[fast_e:], S_BLK, S_T),
    ])
    zero_rows = jnp.zeros((ROWS_PT, D), jnp.float32)

    e1 = jnp.reshape(eps1, (1,))
    e2 = jnp.reshape(eps2, (1,))
    vecs = [v.reshape(1, D) for v in (b1a, g1, be1, b1b, b2a, g2, be2, b2b)]
    b1a_, g1_, be1_, b1b_, b2a_, g2_, be2_, b2b_ = vecs

    sc_agg = _get_sc_agg()
    parts1 = sc_agg(x, blk_all, zero_rows)
    h1 = _tc_layer(e1, x, parts1, W1a.T, b1a_, g1_, be1_, W1b.T, b1b_,
                   final_act=True)
    parts2 = sc_agg(h1, blk_all, zero_rows)
    h2 = _tc_layer(e2, h1, parts2, W2a.T, b2a_, g2_, be2_, W2b.T, b2b_,
                   final_act=False)
    return h2
